# Initial kernel scaffold; baseline (speedup 1.0000x reference)
import jax
import jax.numpy as jnp
from jax.experimental import pallas as pl


def kernel(z_e, W):
    raise NotImplementedError("write your pallas kernel here")


# speedup vs baseline: 2.4515x; 2.4515x over previous
import jax
import jax.numpy as jnp
from jax.experimental import pallas as pl


def _vq_body(z_ref, w_ref, out_ref, loss_ref):
    e = z_ref[0]
    w = w_ref[...]
    wsq = jnp.sum(w * w, axis=1, keepdims=True)
    m = jax.lax.dot_general(w, e, (((1,), (0,)), ((), ())),
                            preferred_element_type=jnp.float32)
    s = wsq - 2.0 * m
    smin = jnp.min(s, axis=0)
    kio = jax.lax.broadcasted_iota(jnp.int32, s.shape, 0)
    amin = jnp.min(jnp.where(s == smin[None, :], kio, jnp.int32(1 << 30)),
                   axis=0)
    onehot = (kio == amin[None, :]).astype(jnp.float32)
    zq = jax.lax.dot_general(w, onehot, (((0,), (0,)), ((), ())),
                             preferred_element_type=jnp.float32)
    out_ref[0] = zq
    zsq = jnp.sum(e * e, axis=0)
    tot = jnp.sum(smin + zsq)
    loss_ref[0] = jnp.full((1, 128), 1.25 * tot / 65536.0, jnp.float32)


def kernel(z_e, W):
    B, C, H, Wd = z_e.shape
    P = H * Wd
    z = z_e.reshape(B, C, P)
    out, loss = pl.pallas_call(
        _vq_body,
        grid=(B,),
        in_specs=[
            pl.BlockSpec((1, C, P), lambda b: (b, 0, 0)),
            pl.BlockSpec((W.shape[0], W.shape[1]), lambda b: (0, 0)),
        ],
        out_specs=[
            pl.BlockSpec((1, C, P), lambda b: (b, 0, 0)),
            pl.BlockSpec((1, 1, 128), lambda b: (b, 0, 0)),
        ],
        out_shape=[
            jax.ShapeDtypeStruct((B, C, P), jnp.float32),
            jax.ShapeDtypeStruct((B, 1, 128), jnp.float32),
        ],
    )(z, W)
    return out.reshape(B, C, H, Wd), loss[:, 0, 0]
